# Initial kernel scaffold; baseline (speedup 1.0000x reference)
import functools

import jax
import jax.numpy as jnp
from jax import lax
from jax.experimental import pallas as pl
from jax.experimental.pallas import tpu as pltpu
from jax.experimental.pallas import tpu_sc as plsc

B = 4096
WLEN = 200
ELEN = 50
DIM = 64
NUM_CLASSES = 16

NC = 2
NS = 16
NW = NC * NS
BPW = B // NW
CB = 4
NCHUNK = BPW // CB


def _sc_gather(word_ids2, entity_ids, word_table, entity_table):
    mesh = plsc.VectorSubcoreMesh(core_axis_name="c", subcore_axis_name="s",
                                  num_cores=NC, num_subcores=NS)

    @functools.partial(
        pl.kernel,
        out_type=(
            jax.ShapeDtypeStruct((B, DIM), jnp.float32),
            jax.ShapeDtypeStruct((B * ELEN, DIM), jnp.float32),
        ),
        mesh=mesh,
        scratch_types=[
            pltpu.VMEM((2 * CB, 100), jnp.int32),
            pltpu.VMEM((CB * WLEN, DIM), jnp.float32),
            pltpu.VMEM((CB, ELEN), jnp.int32),
            pltpu.VMEM((CB * ELEN, DIM), jnp.float32),
            pltpu.VMEM((CB, DIM), jnp.float32),
            pltpu.SemaphoreType.DMA,
        ],
    )
    def k(wid_hbm, eid_hbm, wtab_hbm, etab_hbm, wsum_hbm, erows_hbm,
          widx_v, wrows_v, eidx_v, erows_v, wsum_v, sem):
        w = lax.axis_index("s") * NC + lax.axis_index("c")

        def chunk(c, carry):
            base = w * BPW + c * CB
            pltpu.sync_copy(wid_hbm.at[pl.ds(base * 2, 2 * CB)], widx_v)
            pltpu.sync_copy(eid_hbm.at[pl.ds(base, CB)], eidx_v)
            cps = []
            for j in range(2 * CB):
                cps.append(pltpu.async_copy(
                    wtab_hbm.at[widx_v.at[j]],
                    wrows_v.at[pl.ds(j * 100, 100)], sem))
            for j in range(CB):
                cps.append(pltpu.async_copy(
                    etab_hbm.at[eidx_v.at[j]],
                    erows_v.at[pl.ds(j * ELEN, ELEN)], sem))
            for cp in cps:
                cp.wait()
            pltpu.sync_copy(erows_v, erows_hbm.at[pl.ds(base * ELEN, CB * ELEN)])

            def rbody(r, accs):
                new = []
                for cc in range(CB):
                    for j in range(DIM // 16):
                        new.append(accs[cc * 4 + j]
                                   + wrows_v[cc * WLEN + r, pl.ds(j * 16, 16)])
                return tuple(new)

            zero = jnp.zeros((16,), jnp.float32)
            accs = lax.fori_loop(0, WLEN, rbody, (zero,) * (CB * 4))
            for cc in range(CB):
                for j in range(DIM // 16):
                    wsum_v[cc, pl.ds(j * 16, 16)] = accs[cc * 4 + j]
            pltpu.sync_copy(wsum_v, wsum_hbm.at[pl.ds(base, CB)])
            return carry

        lax.fori_loop(0, NCHUNK, chunk, 0)

    return k(word_ids2, entity_ids, word_table, entity_table)


def _tc_body(ws_ref, ev_ref, wid_ref, eid_ref, pp_ref, asc_ref, owt_ref,
             ob_ref, o_ref):
    ws = ws_ref[...]
    ev = ev_ref[...]
    wn = jnp.maximum(jnp.sqrt(jnp.sum(ws * ws, axis=1, keepdims=True)), 1e-12)
    wnv = ws / wn
    en = jnp.maximum(jnp.sqrt(jnp.sum(ev * ev, axis=2, keepdims=True)), 1e-12)
    env = ev / en
    cos = jnp.sum(wnv[:, None, :] * env, axis=2)
    logits = pp_ref[...] * asc_ref[0] + cos * asc_ref[1] + asc_ref[2]
    logits = jnp.where(eid_ref[...] == 0, -1e32, logits)
    m = jnp.max(logits, axis=1, keepdims=True)
    e = jnp.exp(logits - m)
    att = e / jnp.sum(e, axis=1, keepdims=True)
    feat = jnp.sum(ev * att[:, :, None], axis=1)
    nz = jnp.sum((wid_ref[...] != 0).astype(jnp.float32), axis=1, keepdims=True)
    feat = feat + ws / nz
    o_ref[...] = (
        jnp.dot(feat, owt_ref[...], preferred_element_type=jnp.float32,
                precision=lax.Precision.HIGHEST)
        + ob_ref[...])


def _tc_dense(word_sum, ent_vecs, word_ids, entity_ids, prior_probs,
              att_scalars, out_wt, out_b2):
    BB = 512
    grid = (B // BB,)
    return pl.pallas_call(
        _tc_body,
        grid=grid,
        in_specs=[
            pl.BlockSpec((BB, DIM), lambda i: (i, 0)),
            pl.BlockSpec((BB, ELEN, DIM), lambda i: (i, 0, 0)),
            pl.BlockSpec((BB, WLEN), lambda i: (i, 0)),
            pl.BlockSpec((BB, ELEN), lambda i: (i, 0)),
            pl.BlockSpec((BB, ELEN), lambda i: (i, 0)),
            pl.BlockSpec(memory_space=pltpu.SMEM),
            pl.BlockSpec((DIM, NUM_CLASSES), lambda i: (0, 0)),
            pl.BlockSpec((1, NUM_CLASSES), lambda i: (0, 0)),
        ],
        out_specs=pl.BlockSpec((BB, NUM_CLASSES), lambda i: (i, 0)),
        out_shape=jax.ShapeDtypeStruct((B, NUM_CLASSES), jnp.float32),
    )(word_sum, ent_vecs, word_ids, entity_ids, prior_probs, att_scalars,
      out_wt, out_b2)


def kernel(word_ids, entity_ids, prior_probs, word_table, entity_table,
           att_w, att_b, out_w, out_b):
    word_ids2 = word_ids.reshape(B * 2, 100)
    word_sum, ent_rows = _sc_gather(word_ids2, entity_ids, word_table,
                                    entity_table)
    ent_vecs = ent_rows.reshape(B, ELEN, DIM)
    att_scalars = jnp.stack([att_w[0, 0], att_w[0, 1], att_b[0]])
    return _tc_dense(word_sum, ent_vecs, word_ids, entity_ids, prior_probs,
                     att_scalars, out_w.T, out_b.reshape(1, NUM_CLASSES))


# speedup vs baseline: 1.4292x; 1.4292x over previous
import functools

import jax
import jax.numpy as jnp
from jax import lax
from jax.experimental import pallas as pl
from jax.experimental.pallas import tpu as pltpu
from jax.experimental.pallas import tpu_sc as plsc

B = 4096
WLEN = 200
ELEN = 50
DIM = 64
NUM_CLASSES = 16

NC = 2
NS = 16
NW = NC * NS
BPW = B // NW
CB = 4
NCHUNK = BPW // CB


def _sc_gather(word_ids2, entity_ids, word_table, entity_table):
    mesh = plsc.VectorSubcoreMesh(core_axis_name="c", subcore_axis_name="s",
                                  num_cores=NC, num_subcores=NS)

    @functools.partial(
        pl.kernel,
        out_type=(
            jax.ShapeDtypeStruct((B, DIM), jnp.float32),
            jax.ShapeDtypeStruct((B * ELEN, DIM), jnp.float32),
        ),
        mesh=mesh,
        scratch_types=[
            pltpu.VMEM((2 * CB, 100), jnp.int32),
            pltpu.VMEM((CB * WLEN, DIM), jnp.float32),
            pltpu.VMEM((CB, ELEN), jnp.int32),
            pltpu.VMEM((CB * ELEN, DIM), jnp.float32),
            pltpu.VMEM((CB, DIM), jnp.float32),
            pltpu.SemaphoreType.DMA,
        ],
        compiler_params=pltpu.CompilerParams(use_tc_tiling_on_sc=False),
    )
    def k(wid_hbm, eid_hbm, wtab_hbm, etab_hbm, wsum_hbm, erows_hbm,
          widx_v, wrows_v, eidx_v, erows_v, wsum_v, sem):
        w = lax.axis_index("s") * NC + lax.axis_index("c")

        def chunk(c, carry):
            base = w * BPW + c * CB
            pltpu.sync_copy(wid_hbm.at[pl.ds(base * 2, 2 * CB)], widx_v)
            pltpu.sync_copy(eid_hbm.at[pl.ds(base, CB)], eidx_v)
            cps = []
            for j in range(2 * CB):
                cps.append(pltpu.async_copy(
                    wtab_hbm.at[widx_v.at[j]],
                    wrows_v.at[pl.ds(j * 100, 100)], sem))
            for j in range(CB):
                cps.append(pltpu.async_copy(
                    etab_hbm.at[eidx_v.at[j]],
                    erows_v.at[pl.ds(j * ELEN, ELEN)], sem))
            for cp in cps:
                cp.wait()
            pltpu.sync_copy(erows_v, erows_hbm.at[pl.ds(base * ELEN, CB * ELEN)])

            def rbody(r, accs):
                new = []
                for cc in range(CB):
                    for j in range(DIM // 16):
                        new.append(accs[cc * 4 + j]
                                   + wrows_v[cc * WLEN + r, pl.ds(j * 16, 16)])
                return tuple(new)

            zero = jnp.zeros((16,), jnp.float32)
            accs = lax.fori_loop(0, WLEN, rbody, (zero,) * (CB * 4))
            for cc in range(CB):
                for j in range(DIM // 16):
                    wsum_v[cc, pl.ds(j * 16, 16)] = accs[cc * 4 + j]
            pltpu.sync_copy(wsum_v, wsum_hbm.at[pl.ds(base, CB)])
            return carry

        lax.fori_loop(0, NCHUNK, chunk, 0)

    return k(word_ids2, entity_ids, word_table, entity_table)


def _tc_body(ws_ref, ev_ref, wid_ref, eid_ref, pp_ref, asc_ref, owt_ref,
             ob_ref, o_ref):
    ws = ws_ref[...]
    ev = ev_ref[...]
    wn = jnp.maximum(jnp.sqrt(jnp.sum(ws * ws, axis=1, keepdims=True)), 1e-12)
    wnv = ws / wn
    en = jnp.maximum(jnp.sqrt(jnp.sum(ev * ev, axis=2, keepdims=True)), 1e-12)
    env = ev / en
    cos = jnp.sum(wnv[:, None, :] * env, axis=2)
    logits = pp_ref[...] * asc_ref[0] + cos * asc_ref[1] + asc_ref[2]
    logits = jnp.where(eid_ref[...] == 0, -1e32, logits)
    m = jnp.max(logits, axis=1, keepdims=True)
    e = jnp.exp(logits - m)
    att = e / jnp.sum(e, axis=1, keepdims=True)
    feat = jnp.sum(ev * att[:, :, None], axis=1)
    nz = jnp.sum((wid_ref[...] != 0).astype(jnp.float32), axis=1, keepdims=True)
    feat = feat + ws / nz
    o_ref[...] = (
        jnp.dot(feat, owt_ref[...], preferred_element_type=jnp.float32,
                precision=lax.Precision.HIGHEST)
        + ob_ref[...])


def _tc_dense(word_sum, ent_vecs, word_ids, entity_ids, prior_probs,
              att_scalars, out_wt, out_b2):
    BB = 256
    grid = (B // BB,)
    return pl.pallas_call(
        _tc_body,
        grid=grid,
        in_specs=[
            pl.BlockSpec((BB, DIM), lambda i: (i, 0)),
            pl.BlockSpec((BB, ELEN, DIM), lambda i: (i, 0, 0)),
            pl.BlockSpec((BB, WLEN), lambda i: (i, 0)),
            pl.BlockSpec((BB, ELEN), lambda i: (i, 0)),
            pl.BlockSpec((BB, ELEN), lambda i: (i, 0)),
            pl.BlockSpec(memory_space=pltpu.SMEM),
            pl.BlockSpec((DIM, NUM_CLASSES), lambda i: (0, 0)),
            pl.BlockSpec((1, NUM_CLASSES), lambda i: (0, 0)),
        ],
        out_specs=pl.BlockSpec((BB, NUM_CLASSES), lambda i: (i, 0)),
        out_shape=jax.ShapeDtypeStruct((B, NUM_CLASSES), jnp.float32),
    )(word_sum, ent_vecs, word_ids, entity_ids, prior_probs, att_scalars,
      out_wt, out_b2)


def kernel(word_ids, entity_ids, prior_probs, word_table, entity_table,
           att_w, att_b, out_w, out_b):
    word_ids2 = word_ids.reshape(B * 2, 100)
    word_sum, ent_rows = _sc_gather(word_ids2, entity_ids, word_table,
                                    entity_table)
    ent_vecs = ent_rows.reshape(B, ELEN, DIM)
    att_scalars = jnp.stack([att_w[0, 0], att_w[0, 1], att_b[0]])
    return _tc_dense(word_sum, ent_vecs, word_ids, entity_ids, prior_probs,
                     att_scalars, out_w.T, out_b.reshape(1, NUM_CLASSES))
